# baseline (speedup 1.0000x reference)
import functools

import jax
import jax.numpy as jnp
from jax import lax
from jax.experimental import pallas as pl
from jax.experimental.pallas import tpu as pltpu
from jax.experimental.pallas import tpu_sc as plsc


def _make_gather(V, D, N):
    info = plsc.get_sparse_core_info()
    NC, NS = info.num_cores, info.num_subcores
    NW = NC * NS
    SB = 128
    per_w = N // NW
    assert N % (NW * SB) == 0
    rows_per_w = per_w // SB
    K = 8
    assert rows_per_w % K == 0 and (per_w // SB) % K == 0
    n_chunks = rows_per_w // K
    CH = K * SB

    mesh = plsc.VectorSubcoreMesh(core_axis_name="c", subcore_axis_name="s")

    @functools.partial(
        pl.kernel,
        mesh=mesh,
        out_type=jax.ShapeDtypeStruct((N, D), jnp.float32),
        scratch_types=[
            pltpu.VMEM((K, SB), jnp.int32),
            pltpu.VMEM((CH, D), jnp.float32),
            pltpu.SemaphoreType.DMA,
        ],
        compiler_params=pltpu.CompilerParams(use_tc_tiling_on_sc=False),
    )
    def gather_k(table_hbm, idx_hbm, out_hbm, idx_v, rows_v, sem):
        wid = lax.axis_index("s") * NC + lax.axis_index("c")

        def body(c, carry):
            row0 = wid * rows_per_w + c * K
            pltpu.sync_copy(idx_hbm.at[pl.ds(row0, K)], idx_v)
            handles = []
            for j in range(K):
                handles.append(
                    pltpu.async_copy(
                        table_hbm.at[idx_v.at[j]],
                        rows_v.at[pl.ds(j * SB, SB)],
                        sem,
                    )
                )
            for h in handles:
                h.wait()
            pltpu.sync_copy(rows_v, out_hbm.at[pl.ds(row0 * SB, CH)])
            return carry

        lax.fori_loop(0, n_chunks, body, 0, unroll=False)

    return gather_k


_RB2 = 512


def _tf_body(e0_ref, e1_ref, e2_ref, e3_ref, i32_ref, out_ref):
    ident = i32_ref[...]
    for s, e in enumerate((e0_ref, e1_ref, e2_ref, e3_ref)):
        out_ref[:, 32 * s:32 * (s + 1)] = lax.dot_general(
            e[...], ident, (((0,), (0,)), ((), ())),
            preferred_element_type=jnp.float32,
        )


def _make_transform(V):
    NB = -(-V // (4 * _RB2))
    NBLK = -(-V // _RB2)
    def im(s):
        return lambda j: (0, jnp.minimum(4 * j + s, NBLK - 1))
    return pl.pallas_call(
        _tf_body,
        grid=(NB,),
        in_specs=[
            pl.BlockSpec((32, _RB2), im(0)),
            pl.BlockSpec((32, _RB2), im(1)),
            pl.BlockSpec((32, _RB2), im(2)),
            pl.BlockSpec((32, _RB2), im(3)),
            pl.BlockSpec((32, 32), lambda j: (0, 0)),
        ],
        out_specs=pl.BlockSpec((_RB2, 128), lambda j: (j, 0)),
        out_shape=jax.ShapeDtypeStruct((NB * _RB2, 128), jnp.float32),
    )


def _remap(i):
    return (i >> 11 << 11) + ((i & 511) << 2) + ((i & 2047) >> 9)


_RB = 1024


def _proj_body_T(h_ref, bd_ref, b_ref, out_ref):
    tt = lax.dot_general(
        bd_ref[...], h_ref[...], (((0,), (1,)), ((), ())),
        preferred_element_type=jnp.float32,
    )
    bcol = b_ref[...]
    for s in range(4):
        out_ref[0, :, s * _RB:(s + 1) * _RB] = tt[32 * s:32 * (s + 1), :] + bcol


def _make_proj_T(L, Bb):
    NJ = Bb // (4 * _RB)
    return pl.pallas_call(
        _proj_body_T,
        grid=(L, NJ),
        in_specs=[
            pl.BlockSpec((_RB, 128), lambda l, j: (l * NJ + j, 0)),
            pl.BlockSpec((128, 128), lambda l, j: (0, 0)),
            pl.BlockSpec((32, 1), lambda l, j: (0, 0)),
        ],
        out_specs=pl.BlockSpec((1, 32, 4 * _RB), lambda l, j: (l, 0, j)),
        out_shape=jax.ShapeDtypeStruct((L, 32, Bb), jnp.float32),
    )


def kernel(x, emb, W, b):
    B, L = x.shape
    V, D = emb.shape
    N = B * L
    embt = emb.T
    t2 = _make_transform(V)(embt, embt, embt, embt, jnp.eye(32, dtype=jnp.float32))
    table = t2.reshape(-1, D)
    idxp = _remap(x.T).reshape(L, B // (4 * _RB), 4, _RB)
    idxp = idxp.transpose(0, 1, 3, 2).reshape(N // 128, 128)
    gathered = _make_gather(table.shape[0], D, N)(table, idxp)
    g128 = gathered.reshape(N // 4, 128)
    bd = jnp.kron(jnp.eye(4, dtype=jnp.float32), W.T)
    outp = _make_proj_T(L, B)(g128, bd, b.reshape(32, 1))
    return outp.transpose(2, 0, 1)


# speedup vs baseline: 1.2334x; 1.2334x over previous
import functools

import jax
import jax.numpy as jnp
from jax import lax
from jax.experimental import pallas as pl
from jax.experimental.pallas import tpu as pltpu
from jax.experimental.pallas import tpu_sc as plsc


def _make_gather(V, D, N):
    info = plsc.get_sparse_core_info()
    NC, NS = info.num_cores, info.num_subcores
    NW = NC * NS
    SB = 128
    per_w = N // NW
    assert N % (NW * SB) == 0
    rows_per_w = per_w // SB
    K = 8
    assert rows_per_w % K == 0 and (per_w // SB) % K == 0
    n_chunks = rows_per_w // K
    CH = K * SB

    mesh = plsc.VectorSubcoreMesh(core_axis_name="c", subcore_axis_name="s")

    @functools.partial(
        pl.kernel,
        mesh=mesh,
        out_type=jax.ShapeDtypeStruct((N, D), jnp.float32),
        scratch_types=[
            pltpu.VMEM((K, SB), jnp.int32),
            pltpu.VMEM((CH, D), jnp.float32),
            pltpu.SemaphoreType.DMA,
        ],
        compiler_params=pltpu.CompilerParams(use_tc_tiling_on_sc=False),
    )
    def gather_k(table_hbm, idx_hbm, out_hbm, idx_v, rows_v, sem):
        wid = lax.axis_index("s") * NC + lax.axis_index("c")

        def body(c, carry):
            row0 = wid * rows_per_w + c * K
            pltpu.sync_copy(idx_hbm.at[pl.ds(row0, K)], idx_v)
            handles = []
            for j in range(K):
                handles.append(
                    pltpu.async_copy(
                        table_hbm.at[idx_v.at[j]],
                        rows_v.at[pl.ds(j * SB, SB)],
                        sem,
                    )
                )
            for h in handles:
                h.wait()
            pltpu.sync_copy(rows_v, out_hbm.at[pl.ds(row0 * SB, CH)])
            return carry

        lax.fori_loop(0, n_chunks, body, 0, unroll=False)

    return gather_k


_RB2 = 2048


def _tf_body(e0_ref, e1_ref, e2_ref, e3_ref, i32_ref, out_ref):
    ident = i32_ref[...]
    for s, e in enumerate((e0_ref, e1_ref, e2_ref, e3_ref)):
        out_ref[:, 32 * s:32 * (s + 1)] = lax.dot_general(
            e[...], ident, (((0,), (0,)), ((), ())),
            preferred_element_type=jnp.float32,
        )


def _make_transform(V):
    NB = -(-V // (4 * _RB2))
    NBLK = -(-V // _RB2)
    def im(s):
        return lambda j: (0, jnp.minimum(4 * j + s, NBLK - 1))
    return pl.pallas_call(
        _tf_body,
        grid=(NB,),
        in_specs=[
            pl.BlockSpec((32, _RB2), im(0)),
            pl.BlockSpec((32, _RB2), im(1)),
            pl.BlockSpec((32, _RB2), im(2)),
            pl.BlockSpec((32, _RB2), im(3)),
            pl.BlockSpec((32, 32), lambda j: (0, 0)),
        ],
        out_specs=pl.BlockSpec((_RB2, 128), lambda j: (j, 0)),
        out_shape=jax.ShapeDtypeStruct((NB * _RB2, 128), jnp.float32),
    )


_RB2_LOG = _RB2.bit_length() - 1


def _remap(i):
    m4 = 4 * _RB2 - 1
    return (i & ~m4) + ((i & (_RB2 - 1)) << 2) + ((i & m4) >> _RB2_LOG)


_RB = 1024


def _proj_body_T(h_ref, bd_ref, b_ref, out_ref):
    tt = lax.dot_general(
        bd_ref[...], h_ref[...], (((0,), (1,)), ((), ())),
        preferred_element_type=jnp.float32,
    )
    bcol = b_ref[...]
    for s in range(4):
        out_ref[0, :, s * _RB:(s + 1) * _RB] = tt[32 * s:32 * (s + 1), :] + bcol


def _make_proj_T(L, Bb):
    NJ = Bb // (4 * _RB)
    return pl.pallas_call(
        _proj_body_T,
        grid=(L, NJ),
        in_specs=[
            pl.BlockSpec((_RB, 128), lambda l, j: (l * NJ + j, 0)),
            pl.BlockSpec((128, 128), lambda l, j: (0, 0)),
            pl.BlockSpec((32, 1), lambda l, j: (0, 0)),
        ],
        out_specs=pl.BlockSpec((1, 32, 4 * _RB), lambda l, j: (l, 0, j)),
        out_shape=jax.ShapeDtypeStruct((L, 32, Bb), jnp.float32),
    )


def kernel(x, emb, W, b):
    B, L = x.shape
    V, D = emb.shape
    N = B * L
    embt = emb.T
    t2 = _make_transform(V)(embt, embt, embt, embt, jnp.eye(32, dtype=jnp.float32))
    table = t2.reshape(-1, D)
    idxp = _remap(x.T).reshape(L, B // (4 * _RB), 4, _RB)
    idxp = idxp.transpose(0, 1, 3, 2).reshape(N // 128, 128)
    gathered = _make_gather(table.shape[0], D, N)(table, idxp)
    g128 = gathered.reshape(N // 4, 128)
    bd = jnp.kron(jnp.eye(4, dtype=jnp.float32), W.T)
    outp = _make_proj_T(L, B)(g128, bd, b.reshape(32, 1))
    return outp.transpose(2, 0, 1)


# speedup vs baseline: 1.2504x; 1.0137x over previous
import functools

import jax
import jax.numpy as jnp
from jax import lax
from jax.experimental import pallas as pl
from jax.experimental.pallas import tpu as pltpu
from jax.experimental.pallas import tpu_sc as plsc


def _make_gather(V, D, N):
    info = plsc.get_sparse_core_info()
    NC, NS = info.num_cores, info.num_subcores
    NW = NC * NS
    SB = 128
    per_w = N // NW
    assert N % (NW * SB) == 0
    rows_per_w = per_w // SB
    K = 8
    assert rows_per_w % K == 0 and (per_w // SB) % K == 0
    n_chunks = rows_per_w // K
    CH = K * SB

    mesh = plsc.VectorSubcoreMesh(core_axis_name="c", subcore_axis_name="s")

    @functools.partial(
        pl.kernel,
        mesh=mesh,
        out_type=jax.ShapeDtypeStruct((N, D), jnp.float32),
        scratch_types=[
            pltpu.VMEM((K, SB), jnp.int32),
            pltpu.VMEM((CH, D), jnp.float32),
            pltpu.SemaphoreType.DMA,
        ],
        compiler_params=pltpu.CompilerParams(use_tc_tiling_on_sc=False),
    )
    def gather_k(table_hbm, idx_hbm, out_hbm, idx_v, rows_v, sem):
        wid = lax.axis_index("s") * NC + lax.axis_index("c")

        def body(c, carry):
            row0 = wid * rows_per_w + c * K
            pltpu.sync_copy(idx_hbm.at[pl.ds(row0, K)], idx_v)
            handles = []
            for j in range(K):
                handles.append(
                    pltpu.async_copy(
                        table_hbm.at[idx_v.at[j]],
                        rows_v.at[pl.ds(j * SB, SB)],
                        sem,
                    )
                )
            for h in handles:
                h.wait()
            pltpu.sync_copy(rows_v, out_hbm.at[pl.ds(row0 * SB, CH)])
            return carry

        lax.fori_loop(0, n_chunks, body, 0, unroll=False)

    return gather_k


_RB2 = 4096


def _tf_body(e0_ref, e1_ref, e2_ref, e3_ref, i32_ref, out_ref):
    ident = i32_ref[...]
    for s, e in enumerate((e0_ref, e1_ref, e2_ref, e3_ref)):
        out_ref[:, 32 * s:32 * (s + 1)] = lax.dot_general(
            e[...], ident, (((0,), (0,)), ((), ())),
            preferred_element_type=jnp.float32,
        )


def _make_transform(V):
    NB = -(-V // (4 * _RB2))
    NBLK = -(-V // _RB2)
    def im(s):
        return lambda j: (0, jnp.minimum(4 * j + s, NBLK - 1))
    return pl.pallas_call(
        _tf_body,
        grid=(NB,),
        in_specs=[
            pl.BlockSpec((32, _RB2), im(0)),
            pl.BlockSpec((32, _RB2), im(1)),
            pl.BlockSpec((32, _RB2), im(2)),
            pl.BlockSpec((32, _RB2), im(3)),
            pl.BlockSpec((32, 32), lambda j: (0, 0)),
        ],
        out_specs=pl.BlockSpec((_RB2, 128), lambda j: (j, 0)),
        out_shape=jax.ShapeDtypeStruct((NB * _RB2, 128), jnp.float32),
    )


_RB2_LOG = _RB2.bit_length() - 1


def _remap(i):
    m4 = 4 * _RB2 - 1
    return (i & ~m4) + ((i & (_RB2 - 1)) << 2) + ((i & m4) >> _RB2_LOG)


_RB = 1024


def _proj_body_T(h_ref, bd_ref, b_ref, out_ref):
    tt = lax.dot_general(
        bd_ref[...], h_ref[...], (((0,), (1,)), ((), ())),
        preferred_element_type=jnp.float32,
    )
    bcol = b_ref[...]
    for s in range(4):
        out_ref[0, :, s * _RB:(s + 1) * _RB] = tt[32 * s:32 * (s + 1), :] + bcol


def _make_proj_T(L, Bb):
    NJ = Bb // (4 * _RB)
    return pl.pallas_call(
        _proj_body_T,
        grid=(L, NJ),
        in_specs=[
            pl.BlockSpec((_RB, 128), lambda l, j: (l * NJ + j, 0)),
            pl.BlockSpec((128, 128), lambda l, j: (0, 0)),
            pl.BlockSpec((32, 1), lambda l, j: (0, 0)),
        ],
        out_specs=pl.BlockSpec((1, 32, 4 * _RB), lambda l, j: (l, 0, j)),
        out_shape=jax.ShapeDtypeStruct((L, 32, Bb), jnp.float32),
    )


def kernel(x, emb, W, b):
    B, L = x.shape
    V, D = emb.shape
    N = B * L
    embt = emb.T
    t2 = _make_transform(V)(embt, embt, embt, embt, jnp.eye(32, dtype=jnp.float32))
    table = t2.reshape(-1, D)
    idxp = _remap(x.T).reshape(L, B // (4 * _RB), 4, _RB)
    idxp = idxp.transpose(0, 1, 3, 2).reshape(N // 128, 128)
    gathered = _make_gather(table.shape[0], D, N)(table, idxp)
    g128 = gathered.reshape(N // 4, 128)
    bd = jnp.kron(jnp.eye(4, dtype=jnp.float32), W.T)
    outp = _make_proj_T(L, B)(g128, bd, b.reshape(32, 1))
    return outp.transpose(2, 0, 1)


# speedup vs baseline: 1.3540x; 1.0829x over previous
import functools

import jax
import jax.numpy as jnp
from jax import lax
from jax.experimental import pallas as pl
from jax.experimental.pallas import tpu as pltpu
from jax.experimental.pallas import tpu_sc as plsc


def _make_gather(V, D, N):
    info = plsc.get_sparse_core_info()
    NC, NS = info.num_cores, info.num_subcores
    NW = NC * NS
    SB = 128
    per_w = N // NW
    assert N % (NW * SB) == 0
    rows_per_w = per_w // SB
    K = 8
    assert rows_per_w % K == 0 and (per_w // SB) % K == 0
    n_chunks = rows_per_w // K
    CH = K * SB

    mesh = plsc.VectorSubcoreMesh(core_axis_name="c", subcore_axis_name="s")

    @functools.partial(
        pl.kernel,
        mesh=mesh,
        out_type=jax.ShapeDtypeStruct((N, D), jnp.float32),
        scratch_types=[
            pltpu.VMEM((K, SB), jnp.int32),
            pltpu.VMEM((CH, D), jnp.float32),
            pltpu.SemaphoreType.DMA,
        ],
        compiler_params=pltpu.CompilerParams(use_tc_tiling_on_sc=False),
    )
    def gather_k(table_hbm, idx_hbm, out_hbm, idx_v, rows_v, sem):
        wid = lax.axis_index("s") * NC + lax.axis_index("c")

        def body(c, carry):
            row0 = wid * rows_per_w + c * K
            pltpu.sync_copy(idx_hbm.at[pl.ds(row0, K)], idx_v)
            handles = []
            for j in range(K):
                handles.append(
                    pltpu.async_copy(
                        table_hbm.at[idx_v.at[j]],
                        rows_v.at[pl.ds(j * SB, SB)],
                        sem,
                    )
                )
            for h in handles:
                h.wait()
            pltpu.sync_copy(rows_v, out_hbm.at[pl.ds(row0 * SB, CH)])
            return carry

        lax.fori_loop(0, n_chunks, body, 0, unroll=False)

    return gather_k


_RB2 = 4096


def _tf_body(e0_ref, e1_ref, e2_ref, e3_ref, i32_ref, out_ref):
    del i32_ref
    for s, e in enumerate((e0_ref, e1_ref, e2_ref, e3_ref)):
        out_ref[:, 32 * s:32 * (s + 1)] = jnp.swapaxes(e[...], 0, 1)


def _make_transform(V):
    NB = -(-V // (4 * _RB2))
    NBLK = -(-V // _RB2)
    def im(s):
        return lambda j: (0, jnp.minimum(4 * j + s, NBLK - 1))
    return pl.pallas_call(
        _tf_body,
        grid=(NB,),
        in_specs=[
            pl.BlockSpec((32, _RB2), im(0)),
            pl.BlockSpec((32, _RB2), im(1)),
            pl.BlockSpec((32, _RB2), im(2)),
            pl.BlockSpec((32, _RB2), im(3)),
            pl.BlockSpec((32, 32), lambda j: (0, 0)),
        ],
        out_specs=pl.BlockSpec((_RB2, 128), lambda j: (j, 0)),
        out_shape=jax.ShapeDtypeStruct((NB * _RB2, 128), jnp.float32),
    )


_RB2_LOG = _RB2.bit_length() - 1


def _remap(i):
    m4 = 4 * _RB2 - 1
    return (i & ~m4) + ((i & (_RB2 - 1)) << 2) + ((i & m4) >> _RB2_LOG)


_RB = 2048


def _proj_body_T(h_ref, bd_ref, b_ref, out_ref):
    tt = lax.dot_general(
        bd_ref[...], h_ref[...], (((0,), (1,)), ((), ())),
        preferred_element_type=jnp.float32,
    )
    bcol = b_ref[...]
    for s in range(4):
        out_ref[0, :, s * _RB:(s + 1) * _RB] = tt[32 * s:32 * (s + 1), :] + bcol


def _make_proj_T(L, Bb):
    NJ = Bb // (4 * _RB)
    return pl.pallas_call(
        _proj_body_T,
        grid=(L, NJ),
        in_specs=[
            pl.BlockSpec((_RB, 128), lambda l, j: (l * NJ + j, 0)),
            pl.BlockSpec((128, 128), lambda l, j: (0, 0)),
            pl.BlockSpec((32, 1), lambda l, j: (0, 0)),
        ],
        out_specs=pl.BlockSpec((1, 32, 4 * _RB), lambda l, j: (l, 0, j)),
        out_shape=jax.ShapeDtypeStruct((L, 32, Bb), jnp.float32),
    )


def kernel(x, emb, W, b):
    B, L = x.shape
    V, D = emb.shape
    N = B * L
    embt = emb.T
    t2 = _make_transform(V)(embt, embt, embt, embt, jnp.eye(32, dtype=jnp.float32))
    table = t2.reshape(-1, D)
    idxp = _remap(x.T).reshape(L, B // (4 * _RB), 4, _RB)
    idxp = idxp.transpose(0, 1, 3, 2).reshape(N // 128, 128)
    gathered = _make_gather(table.shape[0], D, N)(table, idxp)
    g128 = gathered.reshape(N // 4, 128)
    bd = jnp.kron(jnp.eye(4, dtype=jnp.float32), W.T)
    outp = _make_proj_T(L, B)(g128, bd, b.reshape(32, 1))
    return outp.transpose(2, 0, 1)


# speedup vs baseline: 1.4440x; 1.0664x over previous
import functools

import jax
import jax.numpy as jnp
from jax import lax
from jax.experimental import pallas as pl
from jax.experimental.pallas import tpu as pltpu
from jax.experimental.pallas import tpu_sc as plsc


def _make_gather(V, D, N):
    info = plsc.get_sparse_core_info()
    NC, NS = info.num_cores, info.num_subcores
    NW = NC * NS
    SB = 128
    per_w = N // NW
    assert N % (NW * SB) == 0
    rows_per_w = per_w // SB
    K = 8
    assert rows_per_w % K == 0 and (per_w // SB) % K == 0
    n_chunks = rows_per_w // K
    CH = K * SB

    mesh = plsc.VectorSubcoreMesh(core_axis_name="c", subcore_axis_name="s")

    @functools.partial(
        pl.kernel,
        mesh=mesh,
        out_type=jax.ShapeDtypeStruct((N, D), jnp.float32),
        scratch_types=[
            pltpu.VMEM((K, SB), jnp.int32),
            pltpu.VMEM((CH, D), jnp.float32),
            pltpu.SemaphoreType.DMA,
        ],
        compiler_params=pltpu.CompilerParams(use_tc_tiling_on_sc=False),
    )
    def gather_k(table_hbm, idx_hbm, out_hbm, idx_v, rows_v, sem):
        wid = lax.axis_index("s") * NC + lax.axis_index("c")

        def body(c, carry):
            row0 = wid * rows_per_w + c * K
            pltpu.sync_copy(idx_hbm.at[pl.ds(row0, K)], idx_v)
            handles = []
            for j in range(K):
                handles.append(
                    pltpu.async_copy(
                        table_hbm.at[idx_v.at[j]],
                        rows_v.at[pl.ds(j * SB, SB)],
                        sem,
                    )
                )
            for h in handles:
                h.wait()
            pltpu.sync_copy(rows_v, out_hbm.at[pl.ds(row0 * SB, CH)])
            return carry

        lax.fori_loop(0, n_chunks, body, 0, unroll=False)

    return gather_k


_RB2 = 8192


def _tf_body(e0_ref, e1_ref, e2_ref, e3_ref, i32_ref, out_ref):
    del i32_ref
    for s, e in enumerate((e0_ref, e1_ref, e2_ref, e3_ref)):
        out_ref[:, 32 * s:32 * (s + 1)] = jnp.swapaxes(e[...], 0, 1)


def _make_transform(V):
    NB = -(-V // (4 * _RB2))
    NBLK = -(-V // _RB2)
    def im(s):
        return lambda j: (0, jnp.minimum(4 * j + s, NBLK - 1))
    return pl.pallas_call(
        _tf_body,
        grid=(NB,),
        in_specs=[
            pl.BlockSpec((32, _RB2), im(0)),
            pl.BlockSpec((32, _RB2), im(1)),
            pl.BlockSpec((32, _RB2), im(2)),
            pl.BlockSpec((32, _RB2), im(3)),
            pl.BlockSpec((32, 32), lambda j: (0, 0)),
        ],
        out_specs=pl.BlockSpec((_RB2, 128), lambda j: (j, 0)),
        out_shape=jax.ShapeDtypeStruct((NB * _RB2, 128), jnp.float32),
    )


_RB2_LOG = _RB2.bit_length() - 1


def _remap(i):
    m4 = 4 * _RB2 - 1
    return (i & ~m4) + ((i & (_RB2 - 1)) << 2) + ((i & m4) >> _RB2_LOG)


_RB = 4096


def _proj_body_T(h_ref, bd_ref, b_ref, out_ref):
    tt = lax.dot_general(
        bd_ref[...], h_ref[...], (((0,), (1,)), ((), ())),
        preferred_element_type=jnp.float32,
    )
    bcol = b_ref[...]
    for s in range(4):
        out_ref[0, :, s * _RB:(s + 1) * _RB] = tt[32 * s:32 * (s + 1), :] + bcol


def _make_proj_T(L, Bb):
    NJ = Bb // (4 * _RB)
    return pl.pallas_call(
        _proj_body_T,
        grid=(L, NJ),
        in_specs=[
            pl.BlockSpec((_RB, 128), lambda l, j: (l * NJ + j, 0)),
            pl.BlockSpec((128, 128), lambda l, j: (0, 0)),
            pl.BlockSpec((32, 1), lambda l, j: (0, 0)),
        ],
        out_specs=pl.BlockSpec((1, 32, 4 * _RB), lambda l, j: (l, 0, j)),
        out_shape=jax.ShapeDtypeStruct((L, 32, Bb), jnp.float32),
    )


def kernel(x, emb, W, b):
    B, L = x.shape
    V, D = emb.shape
    N = B * L
    embt = emb.T
    t2 = _make_transform(V)(embt, embt, embt, embt, jnp.eye(32, dtype=jnp.float32))
    table = t2.reshape(-1, D)
    idxp = _remap(x.T).reshape(L, B // (4 * _RB), 4, _RB)
    idxp = idxp.transpose(0, 1, 3, 2).reshape(N // 128, 128)
    gathered = _make_gather(table.shape[0], D, N)(table, idxp)
    g128 = gathered.reshape(N // 4, 128)
    bd = jnp.kron(jnp.eye(4, dtype=jnp.float32), W.T)
    outp = _make_proj_T(L, B)(g128, bd, b.reshape(32, 1))
    return outp.transpose(2, 0, 1)
